# baseline (speedup 1.0000x reference)
import functools

import jax
import jax.numpy as jnp
from jax import lax
from jax.experimental import pallas as pl
from jax.experimental.pallas import tpu as pltpu

_PLACEHOLDER_TOKEN = 42
_B = 1024
_N = 77
_D = 768
_ROWS = _B * _N
_TROW = 616
_TLANE = 128
_SLABS = 16
_SLAB_ROWS = _ROWS // _SLABS


def _body(tok_hbm, emb_hbm, ph_hbm, out_hbm, tok_v, copy_sems, tok_sem, fix_sem):
    copies = [
        pltpu.make_async_copy(
            emb_hbm.at[pl.ds(s * _SLAB_ROWS, _SLAB_ROWS), :],
            out_hbm.at[pl.ds(s * _SLAB_ROWS, _SLAB_ROWS), :],
            copy_sems.at[s],
        )
        for s in range(_SLABS)
    ]
    for c in copies:
        c.start()

    tok_cp = pltpu.make_async_copy(tok_hbm, tok_v, tok_sem)
    tok_cp.start()
    tok_cp.wait()
    mask = tok_v[...] == _PLACEHOLDER_TOKEN
    cnt = jnp.sum(jnp.where(mask, 1, 0))

    for c in copies:
        c.wait()

    @pl.when(cnt > 0)
    def _():
        def fix(i, carry):
            m = tok_v[...] == _PLACEHOLDER_TOKEN
            rid = lax.broadcasted_iota(jnp.int32, (_TROW, _TLANE), 0)
            lid = lax.broadcasted_iota(jnp.int32, (_TROW, _TLANE), 1)
            flat2 = rid * _TLANE + lid
            flat = jnp.min(jnp.where(m, flat2, jnp.int32(2**30)))
            dma = pltpu.make_async_copy(ph_hbm.at[0], out_hbm.at[flat], fix_sem)
            dma.start()
            dma.wait()
            r = flat // _TLANE
            l = flat - r * _TLANE
            row = tok_v[pl.ds(r, 1), :]
            lvec = lax.broadcasted_iota(jnp.int32, (1, _TLANE), 1)
            tok_v[pl.ds(r, 1), :] = jnp.where(lvec == l, jnp.int32(0), row)
            return carry

        lax.fori_loop(0, cnt, fix, 0)


@jax.jit
def _scatter_copy(tok2d, emb, ph):
    return pl.pallas_call(
        _body,
        grid=(),
        in_specs=[
            pl.BlockSpec(memory_space=pltpu.MemorySpace.HBM),
            pl.BlockSpec(memory_space=pltpu.MemorySpace.HBM),
            pl.BlockSpec(memory_space=pltpu.MemorySpace.HBM),
        ],
        out_specs=pl.BlockSpec(memory_space=pltpu.MemorySpace.HBM),
        out_shape=jax.ShapeDtypeStruct((_ROWS, _D), jnp.float32),
        scratch_shapes=[
            pltpu.VMEM((_TROW, _TLANE), jnp.int32),
            pltpu.SemaphoreType.DMA((_SLABS,)),
            pltpu.SemaphoreType.DMA,
            pltpu.SemaphoreType.DMA,
        ],
    )(tok2d, emb, ph)


def kernel(reference_img, tokenized_text, embedded_text, placeholder_embedding):
    tok = tokenized_text.transpose(1, 0).reshape(_TROW, _TLANE)
    emb = embedded_text.transpose(1, 0, 2).reshape(_ROWS, _D)
    out = _scatter_copy(tok, emb, placeholder_embedding)
    return out.reshape(_N, _B, _D).transpose(1, 0, 2)


# speedup vs baseline: 48.4528x; 48.4528x over previous
import functools

import jax
import jax.numpy as jnp
from jax import lax
from jax.experimental import pallas as pl
from jax.experimental.pallas import tpu as pltpu

_PLACEHOLDER_TOKEN = 42
_B = 1024
_N = 77
_D = 768
_ROWS = _B * _N
_TROW = 616
_TLANE = 128


def _body(tok_hbm, emb_alias, ph_hbm, out_hbm, tok_v, tok_sem, fix_sem):
    del emb_alias
    tok_cp = pltpu.make_async_copy(tok_hbm, tok_v, tok_sem)
    tok_cp.start()
    tok_cp.wait()
    mask = tok_v[...] == _PLACEHOLDER_TOKEN
    cnt = jnp.sum(jnp.where(mask, 1, 0))

    @pl.when(cnt > 0)
    def _():
        def fix(i, carry):
            m = tok_v[...] == _PLACEHOLDER_TOKEN
            rid = lax.broadcasted_iota(jnp.int32, (_TROW, _TLANE), 0)
            lid = lax.broadcasted_iota(jnp.int32, (_TROW, _TLANE), 1)
            flat2 = rid * _TLANE + lid
            flat = jnp.min(jnp.where(m, flat2, jnp.int32(2**30)))
            dma = pltpu.make_async_copy(ph_hbm.at[0], out_hbm.at[flat], fix_sem)
            dma.start()
            dma.wait()
            r = flat // _TLANE
            l = flat - r * _TLANE
            row = tok_v[pl.ds(r, 1), :]
            lvec = lax.broadcasted_iota(jnp.int32, (1, _TLANE), 1)
            tok_v[pl.ds(r, 1), :] = jnp.where(lvec == l, jnp.int32(0), row)
            return carry

        lax.fori_loop(0, cnt, fix, 0)


@jax.jit
def _scatter_overwrite(tok2d, emb, ph):
    return pl.pallas_call(
        _body,
        grid=(),
        in_specs=[
            pl.BlockSpec(memory_space=pltpu.MemorySpace.HBM),
            pl.BlockSpec(memory_space=pltpu.MemorySpace.HBM),
            pl.BlockSpec(memory_space=pltpu.MemorySpace.HBM),
        ],
        out_specs=pl.BlockSpec(memory_space=pltpu.MemorySpace.HBM),
        out_shape=jax.ShapeDtypeStruct((_ROWS, _D), jnp.float32),
        input_output_aliases={1: 0},
        scratch_shapes=[
            pltpu.VMEM((_TROW, _TLANE), jnp.int32),
            pltpu.SemaphoreType.DMA,
            pltpu.SemaphoreType.DMA,
        ],
    )(tok2d, emb, ph)


def kernel(reference_img, tokenized_text, embedded_text, placeholder_embedding):
    tok = tokenized_text.transpose(1, 0).reshape(_TROW, _TLANE)
    emb = embedded_text.transpose(1, 0, 2).reshape(_ROWS, _D)
    out = _scatter_overwrite(tok, emb, placeholder_embedding)
    return out.reshape(_N, _B, _D).transpose(1, 0, 2)


# speedup vs baseline: 48.7249x; 1.0056x over previous
import jax
import jax.numpy as jnp
from jax import lax
from jax.experimental import pallas as pl
from jax.experimental.pallas import tpu as pltpu

_PLACEHOLDER_TOKEN = 42
_B = 1024
_N = 77
_D = 768
_ROWS = _B * _N


def _body(tok_hbm, emb_alias, ph_hbm, out_hbm, tok_v, tok_sem, fix_sem):
    del emb_alias
    tok_cp = pltpu.make_async_copy(tok_hbm, tok_v, tok_sem)
    tok_cp.start()
    tok_cp.wait()
    mask = tok_v[...] == _PLACEHOLDER_TOKEN
    cnt = jnp.sum(jnp.where(mask, 1, 0))

    @pl.when(cnt > 0)
    def _():
        def fix(i, carry):
            m = tok_v[...] == _PLACEHOLDER_TOKEN
            nid = lax.broadcasted_iota(jnp.int32, (_N, _B), 0)
            bid = lax.broadcasted_iota(jnp.int32, (_N, _B), 1)
            flat2 = nid * _B + bid
            flat = jnp.min(jnp.where(m, flat2, jnp.int32(2**30)))
            dma = pltpu.make_async_copy(ph_hbm.at[0], out_hbm.at[flat], fix_sem)
            dma.start()
            dma.wait()
            n = flat // _B
            b = flat - n * _B
            row = tok_v[pl.ds(n, 1), :]
            bvec = lax.broadcasted_iota(jnp.int32, (1, _B), 1)
            tok_v[pl.ds(n, 1), :] = jnp.where(bvec == b, jnp.int32(0), row)
            return carry

        lax.fori_loop(0, cnt, fix, 0)


@jax.jit
def _scatter_overwrite(tok_nb, emb, ph):
    return pl.pallas_call(
        _body,
        grid=(),
        in_specs=[
            pl.BlockSpec(memory_space=pltpu.MemorySpace.HBM),
            pl.BlockSpec(memory_space=pltpu.MemorySpace.HBM),
            pl.BlockSpec(memory_space=pltpu.MemorySpace.HBM),
        ],
        out_specs=pl.BlockSpec(memory_space=pltpu.MemorySpace.HBM),
        out_shape=jax.ShapeDtypeStruct((_ROWS, _D), jnp.float32),
        input_output_aliases={1: 0},
        scratch_shapes=[
            pltpu.VMEM((_N, _B), jnp.int32),
            pltpu.SemaphoreType.DMA,
            pltpu.SemaphoreType.DMA,
        ],
    )(tok_nb, emb, ph)


def kernel(reference_img, tokenized_text, embedded_text, placeholder_embedding):
    tok = tokenized_text.transpose(1, 0)
    emb = embedded_text.transpose(1, 0, 2).reshape(_ROWS, _D)
    out = _scatter_overwrite(tok, emb, placeholder_embedding)
    return out.reshape(_N, _B, _D).transpose(1, 0, 2)


# speedup vs baseline: 48.7264x; 1.0000x over previous
import jax
import jax.numpy as jnp
from jax import lax
from jax.experimental import pallas as pl
from jax.experimental.pallas import tpu as pltpu

_PLACEHOLDER_TOKEN = 42
_B = 1024
_N = 77
_D = 768
_ROWS = _B * _N


def _body(tok_hbm, emb_alias, ph_hbm, out_hbm, tok_v, tok_sem, fix_sem):
    del emb_alias
    tok_cp = pltpu.make_async_copy(tok_hbm, tok_v, tok_sem)
    tok_cp.start()
    tok_cp.wait()
    any0 = jnp.any(tok_v[...] == _PLACEHOLDER_TOKEN)

    def fix(has_match):
        m = tok_v[...] == _PLACEHOLDER_TOKEN
        nid = lax.broadcasted_iota(jnp.int32, (_N, _B), 0)
        bid = lax.broadcasted_iota(jnp.int32, (_N, _B), 1)
        flat2 = nid * _B + bid
        flat = jnp.min(jnp.where(m, flat2, jnp.int32(2**30)))
        dma = pltpu.make_async_copy(ph_hbm.at[0], out_hbm.at[flat], fix_sem)
        dma.start()
        dma.wait()
        n = flat // _B
        b = flat - n * _B
        row = tok_v[pl.ds(n, 1), :]
        bvec = lax.broadcasted_iota(jnp.int32, (1, _B), 1)
        tok_v[pl.ds(n, 1), :] = jnp.where(bvec == b, jnp.int32(0), row)
        return jnp.any(tok_v[...] == _PLACEHOLDER_TOKEN)

    lax.while_loop(lambda has_match: has_match, fix, any0)


@jax.jit
def _scatter_overwrite(tok_nb, emb, ph):
    return pl.pallas_call(
        _body,
        grid=(),
        in_specs=[
            pl.BlockSpec(memory_space=pltpu.MemorySpace.HBM),
            pl.BlockSpec(memory_space=pltpu.MemorySpace.HBM),
            pl.BlockSpec(memory_space=pltpu.MemorySpace.HBM),
        ],
        out_specs=pl.BlockSpec(memory_space=pltpu.MemorySpace.HBM),
        out_shape=jax.ShapeDtypeStruct((_ROWS, _D), jnp.float32),
        input_output_aliases={1: 0},
        scratch_shapes=[
            pltpu.VMEM((_N, _B), jnp.int32),
            pltpu.SemaphoreType.DMA,
            pltpu.SemaphoreType.DMA,
        ],
    )(tok_nb, emb, ph)


def kernel(reference_img, tokenized_text, embedded_text, placeholder_embedding):
    tok = tokenized_text.transpose(1, 0)
    emb = embedded_text.transpose(1, 0, 2).reshape(_ROWS, _D)
    out = _scatter_overwrite(tok, emb, placeholder_embedding)
    return out.reshape(_N, _B, _D).transpose(1, 0, 2)
